# baseline (speedup 1.0000x reference)
import jax
import jax.numpy as jnp
import numpy as np
from jax.experimental import pallas as pl
from jax.experimental.pallas import tpu as pltpu

_VMEM_LIMIT = 100 * 1024 * 1024

_KH1 = np.arange(5)[:, None, None, None]
_IH1 = np.arange(28)[None, :, None, None]
_RH1 = np.arange(2)[None, None, :, None]
_PH1 = np.arange(12)[None, None, None, :]
_U1 = (_IH1 == 2 * _PH1 + _RH1 + _KH1).astype(np.float32)

_KH2 = np.arange(3)[:, None, None]
_IH2 = np.arange(12)[None, :, None]
_OH2 = np.arange(10)[None, None, :]
_U2 = (_IH2 == _OH2 + _KH2).astype(np.float32)


def _build_a1f(w1):
    w1k = w1.reshape(10, 5, 5)
    a = jnp.einsum("hirp,wjsq,ohw->ijrspoq", _U1, _U1, w1k)
    return a.reshape(784, 5760)


def _build_a2f(w2):
    w2k = w2.reshape(20, 10, 3, 3)
    a = jnp.einsum("hio,wjp,nchw->icjnop", _U2, _U2, w2k)
    return a.reshape(1440, 2000)


def _lenet_kernel(x_ref, a1f_ref, b1c_ref, a2f_ref, b2c_ref, wf1_ref, bf1_ref,
                  wf2_ref, bf2_ref, o_ref):
    t = jnp.dot(x_ref[...], a1f_ref[...], preferred_element_type=jnp.float32)
    t = jnp.maximum(t[:, :2880], t[:, 2880:])
    t = jnp.maximum(t[:, :1440], t[:, 1440:])
    t = jnp.maximum(t + b1c_ref[...], 0.0)
    u = jnp.dot(t, a2f_ref[...], preferred_element_type=jnp.float32)
    u = jnp.maximum(u + b2c_ref[...], 0.0)
    h = jnp.dot(u, wf1_ref[...], preferred_element_type=jnp.float32)
    h = jnp.maximum(h + bf1_ref[...], 0.0)
    logits = jnp.dot(h, wf2_ref[...], preferred_element_type=jnp.float32)
    logits = logits + bf2_ref[...]
    m = jnp.max(logits, axis=-1, keepdims=True)
    s = logits - m
    lse = jnp.log(jnp.sum(jnp.exp(s), axis=-1, keepdims=True))
    o_ref[...] = (s - lse).astype(o_ref.dtype)


def kernel(w1, b1, w2, b2, wf1t, bf1, wf2t, bf2, x):
    batch = x.shape[0]
    tb = 128 if batch % 128 == 0 else batch
    xf = x.reshape(batch, 28 * 28)
    a1f = _build_a1f(w1)
    a2f = _build_a2f(w2)
    b1c = jnp.tile(jnp.repeat(b1.reshape(10), 12), 12).reshape(1, 1440)
    b2c = jnp.repeat(b2.reshape(20), 100).reshape(1, 2000)
    cost = pl.CostEstimate(
        flops=2 * batch * (784 * 5760 + 1440 * 2000 + 2000 * 500 + 500 * 10),
        transcendentals=batch * 11,
        bytes_accessed=4 * (xf.size + batch * 10 + a1f.size + a2f.size
                            + wf1t.size + wf2t.size),
    )
    return pl.pallas_call(
        _lenet_kernel,
        out_shape=jax.ShapeDtypeStruct((batch, 10), x.dtype),
        grid=(batch // tb,),
        in_specs=[
            pl.BlockSpec((tb, 784), lambda i: (i, 0)),
            pl.BlockSpec((784, 5760), lambda i: (0, 0)),
            pl.BlockSpec((1, 1440), lambda i: (0, 0)),
            pl.BlockSpec((1440, 2000), lambda i: (0, 0)),
            pl.BlockSpec((1, 2000), lambda i: (0, 0)),
            pl.BlockSpec((2000, 500), lambda i: (0, 0)),
            pl.BlockSpec((1, 500), lambda i: (0, 0)),
            pl.BlockSpec((500, 10), lambda i: (0, 0)),
            pl.BlockSpec((1, 10), lambda i: (0, 0)),
        ],
        out_specs=pl.BlockSpec((tb, 10), lambda i: (i, 0)),
        compiler_params=pltpu.CompilerParams(
            dimension_semantics=("parallel",),
            vmem_limit_bytes=_VMEM_LIMIT,
        ),
        cost_estimate=cost,
    )(xf, a1f, b1c, a2f, b2c, wf1t, bf1, wf2t, bf2)
